# speedup vs baseline: 3.9039x; 3.9039x over previous
import functools

import jax
import jax.numpy as jnp
from jax import lax
from jax.experimental import pallas as pl
from jax.experimental.pallas import tpu as pltpu
from jax.experimental.pallas import tpu_sc as plsc

N = 320000
D = 128
S = 1024

NC = 2
NS = 16
NW = NC * NS
ROWS_PER_W = N // NW
CHUNK = 80
NCHUNK = ROWS_PER_W // CHUNK
ROWS_PER_TILE_OUT = S // NS


def _sc_body(x_hbm, batch_hbm, out_hbm, xbuf, idxbuf, zbuf, acc):
    c = lax.axis_index("c")
    s = lax.axis_index("s")
    wid = c * NS + s
    base = wid * ROWS_PER_W

    def zrow(i, _):
        for j in range(D // 16):
            zbuf[i, pl.ds(j * 16, 16)] = jnp.zeros((16,), jnp.float32)
        return 0
    lax.fori_loop(0, ROWS_PER_TILE_OUT, zrow, 0)
    pltpu.sync_copy(zbuf, acc.at[pl.ds(s * ROWS_PER_TILE_OUT, ROWS_PER_TILE_OUT)])
    plsc.subcore_barrier()

    def body(i, _):
        off = base + i * CHUNK
        pltpu.sync_copy(x_hbm.at[pl.ds(off, CHUNK)], xbuf)
        pltpu.sync_copy(batch_hbm.at[pl.ds(off, CHUNK)], idxbuf)
        pltpu.sync_copy(xbuf, acc.at[idxbuf], add=True)
        return 0
    lax.fori_loop(0, NCHUNK, body, 0)

    plsc.subcore_barrier()
    pltpu.sync_copy(
        acc.at[pl.ds(s * ROWS_PER_TILE_OUT, ROWS_PER_TILE_OUT)],
        out_hbm.at[c, pl.ds(s * ROWS_PER_TILE_OUT, ROWS_PER_TILE_OUT)],
    )


_sc_stage = functools.partial(
    pl.kernel,
    out_type=jax.ShapeDtypeStruct((NC, S, D), jnp.float32),
    mesh=plsc.VectorSubcoreMesh(core_axis_name="c", subcore_axis_name="s"),
    scratch_types=[
        pltpu.VMEM((CHUNK, D), jnp.float32),
        pltpu.VMEM((CHUNK,), jnp.int32),
        pltpu.VMEM((ROWS_PER_TILE_OUT, D), jnp.float32),
        pltpu.VMEM_SHARED((S, D), jnp.float32),
    ],
)(_sc_body)


def _add_body(p_ref, o_ref):
    o_ref[...] = p_ref[0] + p_ref[1]


def kernel(x, batch):
    partials = _sc_stage(x, batch)
    out = pl.pallas_call(
        _add_body,
        out_shape=jax.ShapeDtypeStruct((S, D), jnp.float32),
    )(partials)
    return out


# speedup vs baseline: 7.2618x; 1.8601x over previous
import functools

import jax
import jax.numpy as jnp
from jax import lax
from jax.experimental import pallas as pl
from jax.experimental.pallas import tpu as pltpu
from jax.experimental.pallas import tpu_sc as plsc

N = 320000
D = 128
S = 1024

NC = 2
NS = 16
NW = NC * NS
ROWS_PER_W = N // NW
CHUNK = 80
NCHUNK = ROWS_PER_W // CHUNK
ROWS_PER_TILE_OUT = S // NS


def _sc_body(x_hbm, batch_hbm, out_hbm, xb0, xb1, ib0, ib1, zbuf, acc,
             sx0, sx1, ss0, ss1):
    xb = (xb0, xb1)
    ib = (ib0, ib1)
    sx = (sx0, sx1)
    ss = (ss0, ss1)

    c = lax.axis_index("c")
    s = lax.axis_index("s")
    wid = c * NS + s
    base = wid * ROWS_PER_W

    def zrow(i, _):
        for j in range(D // 16):
            zbuf[i, pl.ds(j * 16, 16)] = jnp.zeros((16,), jnp.float32)
        return 0
    lax.fori_loop(0, ROWS_PER_TILE_OUT, zrow, 0)
    pltpu.sync_copy(zbuf, acc.at[pl.ds(s * ROWS_PER_TILE_OUT, ROWS_PER_TILE_OUT)])
    plsc.subcore_barrier()

    def start_load(chunk, b):
        off = base + chunk * CHUNK
        pltpu.async_copy(x_hbm.at[pl.ds(off, CHUNK)], xb[b], sx[b])
        pltpu.async_copy(batch_hbm.at[pl.ds(off, CHUNK)], ib[b], sx[b])

    def wait_load(chunk, b):
        off = base + chunk * CHUNK
        pltpu.make_async_copy(x_hbm.at[pl.ds(off, CHUNK)], xb[b], sx[b]).wait()
        pltpu.make_async_copy(batch_hbm.at[pl.ds(off, CHUNK)], ib[b], sx[b]).wait()

    def start_scatter(b):
        pltpu.async_copy(xb[b], acc.at[ib[b]], ss[b], add=True)

    def wait_scatter(b):
        pltpu.make_async_copy(xb[b], acc.at[ib[b]], ss[b]).wait()

    start_load(0, 0)

    def outer(k, _):
        for b in range(2):
            ch = 2 * k + b
            nb = 1 - b

            @pl.when(ch >= 1)
            def _():
                wait_scatter(nb)
            start_load(ch + 1, nb)
            wait_load(ch, b)
            start_scatter(b)
        return 0

    lax.fori_loop(0, (NCHUNK - 1) // 2, outer, 0)

    wait_scatter(1)
    wait_load(NCHUNK - 1, 0)
    start_scatter(0)
    wait_scatter(0)

    plsc.subcore_barrier()
    pltpu.sync_copy(
        acc.at[pl.ds(s * ROWS_PER_TILE_OUT, ROWS_PER_TILE_OUT)],
        out_hbm.at[c, pl.ds(s * ROWS_PER_TILE_OUT, ROWS_PER_TILE_OUT)],
    )


_sc_stage = functools.partial(
    pl.kernel,
    out_type=jax.ShapeDtypeStruct((NC, S, D), jnp.float32),
    mesh=plsc.VectorSubcoreMesh(core_axis_name="c", subcore_axis_name="s"),
    scratch_types=[
        pltpu.VMEM((CHUNK, D), jnp.float32),
        pltpu.VMEM((CHUNK, D), jnp.float32),
        pltpu.VMEM((CHUNK,), jnp.int32),
        pltpu.VMEM((CHUNK,), jnp.int32),
        pltpu.VMEM((ROWS_PER_TILE_OUT, D), jnp.float32),
        pltpu.VMEM_SHARED((S, D), jnp.float32),
        pltpu.SemaphoreType.DMA,
        pltpu.SemaphoreType.DMA,
        pltpu.SemaphoreType.DMA,
        pltpu.SemaphoreType.DMA,
    ],
)(_sc_body)


def _add_body(p_ref, o_ref):
    o_ref[...] = p_ref[0] + p_ref[1]


def kernel(x, batch):
    partials = _sc_stage(x, batch)
    out = pl.pallas_call(
        _add_body,
        out_shape=jax.ShapeDtypeStruct((S, D), jnp.float32),
    )(partials)
    return out
